# baseline (speedup 1.0000x reference)
import functools
import jax
import jax.numpy as jnp
from jax import lax
from jax.experimental import pallas as pl
from jax.experimental.pallas import tpu as pltpu
from jax.experimental.pallas import tpu_sc as plsc

N_NODES = 10000
N_PAD = 10240
N_EDGES = 320000
E_PAD = 327680
NC = 2
NS = 16
CH = 128
ROWS_PER_TILE = N_PAD // NS
ZROWS = 32
DEG_W = 128
M_BLK = 400


def _mesh():
    return plsc.VectorSubcoreMesh(core_axis_name="c", subcore_axis_name="s")


def _zero_vmem(ref, nrows, width):
    zero16 = jnp.zeros((16,), jnp.float32)

    def body(i, _):
        r = i // (width // 16)
        f = i % (width // 16)
        ref[r, pl.ds(f * 16, 16)] = zero16
        return 0

    lax.fori_loop(0, nrows * (width // 16), body, 0)


def _deg_body(dst_hbm, out_hbm, didx, ones_v, zbuf, acc):
    c = lax.axis_index("c")
    s = lax.axis_index("s")
    wid = c * NS + s
    e_per = E_PAD // (NC * NS)
    n_chunks = e_per // CH

    one16 = jnp.ones((16,), jnp.float32)

    def fill(i, _):
        r = i // (DEG_W // 16)
        f = i % (DEG_W // 16)
        ones_v[r, pl.ds(f * 16, 16)] = one16
        return 0

    lax.fori_loop(0, CH * (DEG_W // 16), fill, 0)
    _zero_vmem(zbuf, ZROWS, DEG_W)

    def zcopy(k, _):
        pltpu.sync_copy(zbuf, acc.at[pl.ds(s * ROWS_PER_TILE + k * ZROWS, ZROWS)])
        return 0

    lax.fori_loop(0, ROWS_PER_TILE // ZROWS, zcopy, 0)
    plsc.subcore_barrier()

    def chunk(j, _):
        off = wid * e_per + j * CH
        pltpu.sync_copy(dst_hbm.at[pl.ds(off, CH)], didx.at[0])
        pltpu.sync_copy(ones_v, acc.at[didx.at[0]], add=True)
        return 0

    lax.fori_loop(0, n_chunks, chunk, 0)
    plsc.subcore_barrier()

    r0 = s * ROWS_PER_TILE

    @pl.when(c == 0)
    def _():
        pltpu.sync_copy(acc.at[pl.ds(r0, ROWS_PER_TILE)],
                        out_hbm.at[0, pl.ds(r0, ROWS_PER_TILE)])

    @pl.when(c == 1)
    def _():
        pltpu.sync_copy(acc.at[pl.ds(r0, ROWS_PER_TILE)],
                        out_hbm.at[1, pl.ds(r0, ROWS_PER_TILE)])


def _sc_degree(dst):
    kern = pl.kernel(
        _deg_body,
        out_type=jax.ShapeDtypeStruct((NC, N_PAD, DEG_W), jnp.float32),
        mesh=_mesh(),
        scratch_types=[
            pltpu.VMEM((2, CH), jnp.int32),
            pltpu.VMEM((CH, DEG_W), jnp.float32),
            pltpu.VMEM((ZROWS, DEG_W), jnp.float32),
            pltpu.VMEM_SHARED((N_PAD, DEG_W), jnp.float32),
        ],
    )
    return kern(dst)


SB = 1


def _agg_body(H, esplit, zt_hbm, src0_hbm, src1_hbm, dst_hbm, out_hbm,
              sidx, didx, rows, zbuf, acc, sem_i, sem_g, sem_s):
    c = lax.axis_index("c")
    s = lax.axis_index("s")
    if esplit:
        e_per = E_PAD // (NC * NS)
        wid = c * NS + s
        base = wid * e_per
    else:
        e_per = E_PAD // NS
        base = s * e_per
    nsb = e_per // (SB * CH)

    _zero_vmem(zbuf, ZROWS, H)

    def zcopy(k, _):
        pltpu.sync_copy(zbuf, acc.at[pl.ds(s * ROWS_PER_TILE + k * ZROWS, ZROWS)])
        return 0

    lax.fori_loop(0, ROWS_PER_TILE // ZROWS, zcopy, 0)
    plsc.subcore_barrier()

    def load_idx(sb, islot, sem):
        off = base + sb * (SB * CH)
        if esplit:
            pltpu.async_copy(src0_hbm.at[pl.ds(off, SB * CH)], sidx.at[islot, 0], sem)
        else:
            @pl.when(c == 0)
            def _():
                pltpu.async_copy(src0_hbm.at[pl.ds(off, SB * CH)], sidx.at[islot, 0], sem)

            @pl.when(c == 1)
            def _():
                pltpu.async_copy(src1_hbm.at[pl.ds(off, SB * CH)], sidx.at[islot, 0], sem)
        for k in range(SB):
            pltpu.async_copy(dst_hbm.at[pl.ds(off + k * CH, CH)],
                             didx.at[islot, k, 0], sem)

    def fire_gathers(islot, rslot):
        for k in range(SB):
            pltpu.async_copy(zt_hbm.at[sidx.at[islot, 0, pl.ds(k * CH, CH)]],
                             rows.at[rslot, k], sem_g)

    def fire_scatters(islot, rslot):
        for k in range(SB):
            pltpu.async_copy(rows.at[rslot, k], acc.at[didx.at[islot, k, 0]],
                             sem_s, add=True)

    def drain_rows(sem, count):
        for _ in range(count):
            pltpu.make_async_copy(zt_hbm.at[pl.ds(0, CH)], rows.at[0, 0], sem).wait()

    def drain_idx(sem, count):
        for _ in range(count):
            pltpu.make_async_copy(src0_hbm.at[pl.ds(0, SB * CH)], sidx.at[0, 0],
                                  sem).wait()
            for k in range(SB):
                pltpu.make_async_copy(dst_hbm.at[pl.ds(0, CH)], didx.at[0, k, 0],
                                      sem).wait()

    load_idx(0, 0, sem_i)
    drain_idx(sem_i, 1)
    fire_gathers(0, 0)

    @pl.when(nsb > 1)
    def _():
        load_idx(1, 1, sem_i)

    def step(g, _):
        rslot = g % 2
        nslot = 1 - rslot
        i_g = g % 3
        i1 = (g + 1) % 3
        i2 = (g + 2) % 3

        drain_rows(sem_g, SB)

        @pl.when(g > 0)
        def _():
            drain_rows(sem_s, SB)

        @pl.when(g + 1 < nsb)
        def _():
            drain_idx(sem_i, 1)
            fire_gathers(i1, nslot)

        fire_scatters(i_g, rslot)

        @pl.when(g + 2 < nsb)
        def _():
            load_idx(g + 2, i2, sem_i)

        return 0

    lax.fori_loop(0, nsb, step, 0)
    drain_rows(sem_s, SB)
    plsc.subcore_barrier()

    r0 = s * ROWS_PER_TILE

    @pl.when(c == 0)
    def _():
        pltpu.sync_copy(acc.at[pl.ds(r0, ROWS_PER_TILE)],
                        out_hbm.at[0, pl.ds(r0, ROWS_PER_TILE)])

    @pl.when(c == 1)
    def _():
        pltpu.sync_copy(acc.at[pl.ds(r0, ROWS_PER_TILE)],
                        out_hbm.at[1, pl.ds(r0, ROWS_PER_TILE)])


def _sc_aggregate(zt_flat, src0, src1, dst, H, esplit):
    kern = pl.kernel(
        functools.partial(_agg_body, H, esplit),
        out_type=jax.ShapeDtypeStruct((NC, N_PAD, H), jnp.float32),
        mesh=_mesh(),
        scratch_types=[
            pltpu.VMEM((3, 1, SB * CH), jnp.int32),
            pltpu.VMEM((3, SB, 1, CH), jnp.int32),
            pltpu.VMEM((2, SB, CH, H), jnp.float32),
            pltpu.VMEM((ZROWS, H), jnp.float32),
            pltpu.VMEM_SHARED((N_PAD, H), jnp.float32),
            pltpu.SemaphoreType.DMA,
            pltpu.SemaphoreType.DMA,
            pltpu.SemaphoreType.DMA,
        ],
    )
    return kern(zt_flat, src0, src1, dst)


def _dinv_from(deg_ref):
    deg = deg_ref[0, :, 0] + deg_ref[1, :, 0] + 1.0
    return lax.rsqrt(deg)[:, None]


def _tc1_body(x_ref, w_ref, deg_ref, o_ref):
    z = jnp.dot(x_ref[...], w_ref[...], preferred_element_type=jnp.float32)
    zt = z * _dinv_from(deg_ref)
    h = zt.shape[1] // 2
    o_ref[0] = zt[:, :h]
    o_ref[1] = zt[:, h:]


def _tc_layer1(x, W1, deg2):
    K = x.shape[1]
    D = W1.shape[1]
    grid = N_NODES // M_BLK
    return pl.pallas_call(
        _tc1_body,
        grid=(grid,),
        in_specs=[
            pl.BlockSpec((M_BLK, K), lambda m: (m, 0)),
            pl.BlockSpec((K, D), lambda m: (0, 0)),
            pl.BlockSpec((2, M_BLK, DEG_W), lambda m: (0, m, 0)),
        ],
        out_specs=pl.BlockSpec((2, M_BLK, D // 2), lambda m: (0, m, 0)),
        out_shape=jax.ShapeDtypeStruct((2, N_NODES, D // 2), jnp.float32),
    )(x, W1, deg2)


def _tc_mid_body(relu, D, D2, split_in, agg_ref, zt_ref, deg_ref, b_ref, w_ref, o_ref):
    dinv = _dinv_from(deg_ref)
    if split_in:
        agg = jnp.concatenate([agg_ref[0], agg_ref[1]], axis=1)
        zt = jnp.concatenate([zt_ref[0], zt_ref[1]], axis=1)
    else:
        agg = (agg_ref[0] + agg_ref[1])[:, :D]
        zt = zt_ref[...][:, :D]
    h = dinv * (agg + zt) + b_ref[...][None, :]
    if relu:
        h = jnp.maximum(h, 0.0)
    z2 = jnp.dot(h, w_ref[...], preferred_element_type=jnp.float32)
    zt2 = z2 * dinv
    if D2 < 128:
        zt2 = jnp.concatenate(
            [zt2, jnp.zeros((zt2.shape[0], 128 - D2), jnp.float32)], axis=1)
    o_ref[...] = zt2


def _tc_mid(agg, zt, deg2, b, W, relu, split_in):
    D = b.shape[0]
    D2 = W.shape[1]
    grid = N_NODES // M_BLK
    if split_in:
        zt_spec = pl.BlockSpec((2, M_BLK, D // 2), lambda m: (0, m, 0))
    else:
        zt_spec = pl.BlockSpec((M_BLK, 128), lambda m: (m, 0))
    return pl.pallas_call(
        functools.partial(_tc_mid_body, relu, D, D2, split_in),
        grid=(grid,),
        in_specs=[
            pl.BlockSpec((2, M_BLK, 128), lambda m: (0, m, 0)),
            zt_spec,
            pl.BlockSpec((2, M_BLK, DEG_W), lambda m: (0, m, 0)),
            pl.BlockSpec((D,), lambda m: (0,)),
            pl.BlockSpec((D, D2), lambda m: (0, 0)),
        ],
        out_specs=pl.BlockSpec((M_BLK, 128), lambda m: (m, 0)),
        out_shape=jax.ShapeDtypeStruct((N_NODES, 128), jnp.float32),
    )(agg, zt, deg2, b, W)


def _tc_final_body(agg_ref, zt_ref, deg_ref, b_ref, wl_ref, bl_ref, h_ref, o_ref):
    D = b_ref.shape[0]
    dinv = _dinv_from(deg_ref)
    agg = (agg_ref[0] + agg_ref[1])[:, :D]
    zt = zt_ref[...][:, :D]
    h3 = dinv * (agg + zt) + b_ref[...][None, :]
    h_ref[...] = h3
    logits = jnp.dot(h3, wl_ref[...], preferred_element_type=jnp.float32)
    o_ref[...] = jax.nn.sigmoid(logits + bl_ref[0])


def _tc_final(agg, zt, deg2, b3, Wl, bl):
    D = b3.shape[0]
    grid = N_NODES // M_BLK
    return pl.pallas_call(
        _tc_final_body,
        grid=(grid,),
        in_specs=[
            pl.BlockSpec((2, M_BLK, 128), lambda m: (0, m, 0)),
            pl.BlockSpec((M_BLK, 128), lambda m: (m, 0)),
            pl.BlockSpec((2, M_BLK, DEG_W), lambda m: (0, m, 0)),
            pl.BlockSpec((D,), lambda m: (0,)),
            pl.BlockSpec((D, 1), lambda m: (0, 0)),
            pl.BlockSpec((1,), lambda m: (0,)),
        ],
        out_specs=[
            pl.BlockSpec((M_BLK, D), lambda m: (m, 0)),
            pl.BlockSpec((M_BLK, 1), lambda m: (m, 0)),
        ],
        out_shape=[
            jax.ShapeDtypeStruct((N_NODES, D), jnp.float32),
            jax.ShapeDtypeStruct((N_NODES, 1), jnp.float32),
        ],
    )(agg, zt, deg2, b3, Wl, bl)


def kernel(x, W1, b1, W2, b2, W3, b3, Wl, bl, edge_index):
    pad = E_PAD - N_EDGES
    src = jnp.concatenate([edge_index[0], jnp.zeros((pad,), jnp.int32)])
    dst = jnp.concatenate([edge_index[1], jnp.full((pad,), N_NODES, jnp.int32)])
    srcN = src + N_NODES
    deg2 = _sc_degree(dst)

    zt1 = _tc_layer1(x, W1, deg2)
    agg1 = _sc_aggregate(zt1.reshape(2 * N_NODES, 128), src, srcN, dst,
                         128, esplit=False)

    zt2 = _tc_mid(agg1, zt1, deg2, b1, W2, relu=True,
                  split_in=True)
    agg2 = _sc_aggregate(zt2, src, srcN, dst, 128, esplit=True)

    zt3 = _tc_mid(agg2, zt2, deg2, b2, W3, relu=False,
                  split_in=False)
    agg3 = _sc_aggregate(zt3, src, srcN, dst, 128, esplit=True)

    h3, out = _tc_final(agg3, zt3, deg2, b3, Wl, bl)
    return (out, h3)


# speedup vs baseline: 1.0796x; 1.0796x over previous
import functools
import jax
import jax.numpy as jnp
from jax import lax
from jax.experimental import pallas as pl
from jax.experimental.pallas import tpu as pltpu
from jax.experimental.pallas import tpu_sc as plsc

N_NODES = 10000
N_PAD = 10240
N_EDGES = 320000
E_PAD = 327680
NC = 2
NS = 16
CH = 128
ROWS_PER_TILE = N_PAD // NS
ZROWS = 32
DEG_W = 128
M_BLK = 400


def _mesh():
    return plsc.VectorSubcoreMesh(core_axis_name="c", subcore_axis_name="s")


def _zero_vmem(ref, nrows, width):
    zero16 = jnp.zeros((16,), jnp.float32)

    def body(i, _):
        r = i // (width // 16)
        f = i % (width // 16)
        ref[r, pl.ds(f * 16, 16)] = zero16
        return 0

    lax.fori_loop(0, nrows * (width // 16), body, 0)


def _deg_body(dst_hbm, out_hbm, didx, ones_v, zbuf, acc):
    c = lax.axis_index("c")
    s = lax.axis_index("s")
    wid = c * NS + s
    e_per = E_PAD // (NC * NS)
    n_chunks = e_per // CH

    one16 = jnp.ones((16,), jnp.float32)

    def fill(i, _):
        r = i // (DEG_W // 16)
        f = i % (DEG_W // 16)
        ones_v[r, pl.ds(f * 16, 16)] = one16
        return 0

    lax.fori_loop(0, CH * (DEG_W // 16), fill, 0)
    _zero_vmem(zbuf, ZROWS, DEG_W)

    def zcopy(k, _):
        pltpu.sync_copy(zbuf, acc.at[pl.ds(s * ROWS_PER_TILE + k * ZROWS, ZROWS)])
        return 0

    lax.fori_loop(0, ROWS_PER_TILE // ZROWS, zcopy, 0)
    plsc.subcore_barrier()

    def chunk(j, _):
        off = wid * e_per + j * CH
        pltpu.sync_copy(dst_hbm.at[pl.ds(off, CH)], didx.at[0])
        pltpu.sync_copy(ones_v, acc.at[didx.at[0]], add=True)
        return 0

    lax.fori_loop(0, n_chunks, chunk, 0)
    plsc.subcore_barrier()

    r0 = s * ROWS_PER_TILE

    @pl.when(c == 0)
    def _():
        pltpu.sync_copy(acc.at[pl.ds(r0, ROWS_PER_TILE)],
                        out_hbm.at[0, pl.ds(r0, ROWS_PER_TILE)])

    @pl.when(c == 1)
    def _():
        pltpu.sync_copy(acc.at[pl.ds(r0, ROWS_PER_TILE)],
                        out_hbm.at[1, pl.ds(r0, ROWS_PER_TILE)])


def _sc_degree(dst):
    kern = pl.kernel(
        _deg_body,
        out_type=jax.ShapeDtypeStruct((NC, N_PAD, DEG_W), jnp.float32),
        mesh=_mesh(),
        scratch_types=[
            pltpu.VMEM((2, CH), jnp.int32),
            pltpu.VMEM((CH, DEG_W), jnp.float32),
            pltpu.VMEM((ZROWS, DEG_W), jnp.float32),
            pltpu.VMEM_SHARED((N_PAD, DEG_W), jnp.float32),
        ],
    )
    return kern(dst)


SB = 1


def _agg_body(H, esplit, zt_hbm, src0_hbm, src1_hbm, dst_hbm, out_hbm,
              sidx, didx, rows, zbuf, acc, sem_i, sem_g, sem_s):
    c = lax.axis_index("c")
    s = lax.axis_index("s")
    if esplit:
        e_per = E_PAD // (NC * NS)
        base = s * (NC * e_per) + c * (SB * CH)
        stride = NC * SB * CH
    else:
        e_per = E_PAD // NS
        base = s * e_per
        stride = SB * CH
    nsb = e_per // (SB * CH)

    _zero_vmem(zbuf, ZROWS, H)

    def zcopy(k, _):
        pltpu.sync_copy(zbuf, acc.at[pl.ds(s * ROWS_PER_TILE + k * ZROWS, ZROWS)])
        return 0

    lax.fori_loop(0, ROWS_PER_TILE // ZROWS, zcopy, 0)
    plsc.subcore_barrier()

    def load_idx(sb, islot, sem):
        off = base + sb * stride
        if esplit:
            pltpu.async_copy(src0_hbm.at[pl.ds(off, SB * CH)], sidx.at[islot, 0], sem)
        else:
            @pl.when(c == 0)
            def _():
                pltpu.async_copy(src0_hbm.at[pl.ds(off, SB * CH)], sidx.at[islot, 0], sem)

            @pl.when(c == 1)
            def _():
                pltpu.async_copy(src1_hbm.at[pl.ds(off, SB * CH)], sidx.at[islot, 0], sem)
        for k in range(SB):
            pltpu.async_copy(dst_hbm.at[pl.ds(off + k * CH, CH)],
                             didx.at[islot, k, 0], sem)

    def fire_gathers(islot, rslot):
        for k in range(SB):
            pltpu.async_copy(zt_hbm.at[sidx.at[islot, 0, pl.ds(k * CH, CH)]],
                             rows.at[rslot, k], sem_g)

    def fire_scatters(islot, rslot):
        for k in range(SB):
            pltpu.async_copy(rows.at[rslot, k], acc.at[didx.at[islot, k, 0]],
                             sem_s, add=True)

    def drain_rows(sem, count):
        for _ in range(count):
            pltpu.make_async_copy(zt_hbm.at[pl.ds(0, CH)], rows.at[0, 0], sem).wait()

    def drain_idx(sem, count):
        for _ in range(count):
            pltpu.make_async_copy(src0_hbm.at[pl.ds(0, SB * CH)], sidx.at[0, 0],
                                  sem).wait()
            for k in range(SB):
                pltpu.make_async_copy(dst_hbm.at[pl.ds(0, CH)], didx.at[0, k, 0],
                                      sem).wait()

    load_idx(0, 0, sem_i)
    drain_idx(sem_i, 1)
    fire_gathers(0, 0)

    @pl.when(nsb > 1)
    def _():
        load_idx(1, 1, sem_i)

    def step(g, _):
        rslot = g % 2
        nslot = 1 - rslot
        i_g = g % 3
        i1 = (g + 1) % 3
        i2 = (g + 2) % 3

        drain_rows(sem_g, SB)

        @pl.when(g > 0)
        def _():
            drain_rows(sem_s, SB)

        @pl.when(g + 1 < nsb)
        def _():
            drain_idx(sem_i, 1)
            fire_gathers(i1, nslot)

        fire_scatters(i_g, rslot)

        @pl.when(g + 2 < nsb)
        def _():
            load_idx(g + 2, i2, sem_i)

        return 0

    lax.fori_loop(0, nsb, step, 0)
    drain_rows(sem_s, SB)
    plsc.subcore_barrier()

    r0 = s * ROWS_PER_TILE

    @pl.when(c == 0)
    def _():
        pltpu.sync_copy(acc.at[pl.ds(r0, ROWS_PER_TILE)],
                        out_hbm.at[0, pl.ds(r0, ROWS_PER_TILE)])

    @pl.when(c == 1)
    def _():
        pltpu.sync_copy(acc.at[pl.ds(r0, ROWS_PER_TILE)],
                        out_hbm.at[1, pl.ds(r0, ROWS_PER_TILE)])


def _sc_aggregate(zt_flat, src0, src1, dst, H, esplit):
    kern = pl.kernel(
        functools.partial(_agg_body, H, esplit),
        out_type=jax.ShapeDtypeStruct((NC, N_PAD, H), jnp.float32),
        mesh=_mesh(),
        scratch_types=[
            pltpu.VMEM((3, 1, SB * CH), jnp.int32),
            pltpu.VMEM((3, SB, 1, CH), jnp.int32),
            pltpu.VMEM((2, SB, CH, H), jnp.float32),
            pltpu.VMEM((ZROWS, H), jnp.float32),
            pltpu.VMEM_SHARED((N_PAD, H), jnp.float32),
            pltpu.SemaphoreType.DMA,
            pltpu.SemaphoreType.DMA,
            pltpu.SemaphoreType.DMA,
        ],
    )
    return kern(zt_flat, src0, src1, dst)


def _dinv_from(deg_ref):
    deg = deg_ref[0, :, 0] + deg_ref[1, :, 0] + 1.0
    return lax.rsqrt(deg)[:, None]


def _tc1_body(x_ref, w_ref, deg_ref, o_ref):
    z = jnp.dot(x_ref[...], w_ref[...], preferred_element_type=jnp.float32)
    zt = z * _dinv_from(deg_ref)
    h = zt.shape[1] // 2
    o_ref[0] = zt[:, :h]
    o_ref[1] = zt[:, h:]


def _tc_layer1(x, W1, deg2):
    K = x.shape[1]
    D = W1.shape[1]
    grid = N_NODES // M_BLK
    return pl.pallas_call(
        _tc1_body,
        grid=(grid,),
        in_specs=[
            pl.BlockSpec((M_BLK, K), lambda m: (m, 0)),
            pl.BlockSpec((K, D), lambda m: (0, 0)),
            pl.BlockSpec((2, M_BLK, DEG_W), lambda m: (0, m, 0)),
        ],
        out_specs=pl.BlockSpec((2, M_BLK, D // 2), lambda m: (0, m, 0)),
        out_shape=jax.ShapeDtypeStruct((2, N_NODES, D // 2), jnp.float32),
    )(x, W1, deg2)


def _tc_mid_body(relu, D, D2, split_in, agg_ref, zt_ref, deg_ref, b_ref, w_ref, o_ref):
    dinv = _dinv_from(deg_ref)
    if split_in:
        agg = jnp.concatenate([agg_ref[0], agg_ref[1]], axis=1)
        zt = jnp.concatenate([zt_ref[0], zt_ref[1]], axis=1)
    else:
        agg = (agg_ref[0] + agg_ref[1])[:, :D]
        zt = zt_ref[...][:, :D]
    h = dinv * (agg + zt) + b_ref[...][None, :]
    if relu:
        h = jnp.maximum(h, 0.0)
    z2 = jnp.dot(h, w_ref[...], preferred_element_type=jnp.float32)
    zt2 = z2 * dinv
    if D2 < 128:
        zt2 = jnp.concatenate(
            [zt2, jnp.zeros((zt2.shape[0], 128 - D2), jnp.float32)], axis=1)
    o_ref[...] = zt2


def _tc_mid(agg, zt, deg2, b, W, relu, split_in):
    D = b.shape[0]
    D2 = W.shape[1]
    grid = N_NODES // M_BLK
    if split_in:
        zt_spec = pl.BlockSpec((2, M_BLK, D // 2), lambda m: (0, m, 0))
    else:
        zt_spec = pl.BlockSpec((M_BLK, 128), lambda m: (m, 0))
    return pl.pallas_call(
        functools.partial(_tc_mid_body, relu, D, D2, split_in),
        grid=(grid,),
        in_specs=[
            pl.BlockSpec((2, M_BLK, 128), lambda m: (0, m, 0)),
            zt_spec,
            pl.BlockSpec((2, M_BLK, DEG_W), lambda m: (0, m, 0)),
            pl.BlockSpec((D,), lambda m: (0,)),
            pl.BlockSpec((D, D2), lambda m: (0, 0)),
        ],
        out_specs=pl.BlockSpec((M_BLK, 128), lambda m: (m, 0)),
        out_shape=jax.ShapeDtypeStruct((N_NODES, 128), jnp.float32),
    )(agg, zt, deg2, b, W)


def _tc_final_body(agg_ref, zt_ref, deg_ref, b_ref, wl_ref, bl_ref, h_ref, o_ref):
    D = b_ref.shape[0]
    dinv = _dinv_from(deg_ref)
    agg = (agg_ref[0] + agg_ref[1])[:, :D]
    zt = zt_ref[...][:, :D]
    h3 = dinv * (agg + zt) + b_ref[...][None, :]
    h_ref[...] = h3
    logits = jnp.dot(h3, wl_ref[...], preferred_element_type=jnp.float32)
    o_ref[...] = jax.nn.sigmoid(logits + bl_ref[0])


def _tc_final(agg, zt, deg2, b3, Wl, bl):
    D = b3.shape[0]
    grid = N_NODES // M_BLK
    return pl.pallas_call(
        _tc_final_body,
        grid=(grid,),
        in_specs=[
            pl.BlockSpec((2, M_BLK, 128), lambda m: (0, m, 0)),
            pl.BlockSpec((M_BLK, 128), lambda m: (m, 0)),
            pl.BlockSpec((2, M_BLK, DEG_W), lambda m: (0, m, 0)),
            pl.BlockSpec((D,), lambda m: (0,)),
            pl.BlockSpec((D, 1), lambda m: (0, 0)),
            pl.BlockSpec((1,), lambda m: (0,)),
        ],
        out_specs=[
            pl.BlockSpec((M_BLK, D), lambda m: (m, 0)),
            pl.BlockSpec((M_BLK, 1), lambda m: (m, 0)),
        ],
        out_shape=[
            jax.ShapeDtypeStruct((N_NODES, D), jnp.float32),
            jax.ShapeDtypeStruct((N_NODES, 1), jnp.float32),
        ],
    )(agg, zt, deg2, b3, Wl, bl)


def kernel(x, W1, b1, W2, b2, W3, b3, Wl, bl, edge_index):
    pad = E_PAD - N_EDGES
    src = jnp.concatenate([edge_index[0], jnp.zeros((pad,), jnp.int32)])
    dst = jnp.concatenate([edge_index[1], jnp.full((pad,), N_NODES, jnp.int32)])
    srcN = src + N_NODES
    deg2 = _sc_degree(dst)

    zt1 = _tc_layer1(x, W1, deg2)
    agg1 = _sc_aggregate(zt1.reshape(2 * N_NODES, 128), src, srcN, dst,
                         128, esplit=False)

    zt2 = _tc_mid(agg1, zt1, deg2, b1, W2, relu=True,
                  split_in=True)
    agg2 = _sc_aggregate(zt2, src, srcN, dst, 128, esplit=True)

    zt3 = _tc_mid(agg2, zt2, deg2, b2, W3, relu=False,
                  split_in=False)
    agg3 = _sc_aggregate(zt3, src, srcN, dst, 128, esplit=True)

    h3, out = _tc_final(agg3, zt3, deg2, b3, Wl, bl)
    return (out, h3)


# speedup vs baseline: 1.1453x; 1.0608x over previous
import functools
import jax
import jax.numpy as jnp
from jax import lax
from jax.experimental import pallas as pl
from jax.experimental.pallas import tpu as pltpu
from jax.experimental.pallas import tpu_sc as plsc

N_NODES = 10000
N_PAD = 10240
N_EDGES = 320000
E_PAD = 327680
NC = 2
NS = 16
CH = 128
ROWS_PER_TILE = N_PAD // NS
ZROWS = 32
DEG_W = 128
M_BLK = 400


def _mesh():
    return plsc.VectorSubcoreMesh(core_axis_name="c", subcore_axis_name="s")


def _zero_vmem(ref, nrows, width):
    zero16 = jnp.zeros((16,), jnp.float32)

    def body(i, _):
        r = i // (width // 16)
        f = i % (width // 16)
        ref[r, pl.ds(f * 16, 16)] = zero16
        return 0

    lax.fori_loop(0, nrows * (width // 16), body, 0)


def _deg_body(dst_hbm, out_hbm, didx, ones_v, zbuf, acc):
    c = lax.axis_index("c")
    s = lax.axis_index("s")
    wid = c * NS + s
    e_per = E_PAD // (NC * NS)
    n_chunks = e_per // CH

    one16 = jnp.ones((16,), jnp.float32)

    def fill(i, _):
        r = i // (DEG_W // 16)
        f = i % (DEG_W // 16)
        ones_v[r, pl.ds(f * 16, 16)] = one16
        return 0

    lax.fori_loop(0, CH * (DEG_W // 16), fill, 0)
    _zero_vmem(zbuf, ZROWS, DEG_W)

    def zcopy(k, _):
        pltpu.sync_copy(zbuf, acc.at[pl.ds(s * ROWS_PER_TILE + k * ZROWS, ZROWS)])
        return 0

    lax.fori_loop(0, ROWS_PER_TILE // ZROWS, zcopy, 0)
    plsc.subcore_barrier()

    def chunk(j, _):
        off = wid * e_per + j * CH
        pltpu.sync_copy(dst_hbm.at[pl.ds(off, CH)], didx.at[0])
        pltpu.sync_copy(ones_v, acc.at[didx.at[0]], add=True)
        return 0

    lax.fori_loop(0, n_chunks, chunk, 0)
    plsc.subcore_barrier()

    r0 = s * ROWS_PER_TILE

    @pl.when(c == 0)
    def _():
        pltpu.sync_copy(acc.at[pl.ds(r0, ROWS_PER_TILE)],
                        out_hbm.at[0, pl.ds(r0, ROWS_PER_TILE)])

    @pl.when(c == 1)
    def _():
        pltpu.sync_copy(acc.at[pl.ds(r0, ROWS_PER_TILE)],
                        out_hbm.at[1, pl.ds(r0, ROWS_PER_TILE)])


def _sc_degree(dst):
    kern = pl.kernel(
        _deg_body,
        out_type=jax.ShapeDtypeStruct((NC, N_PAD, DEG_W), jnp.float32),
        mesh=_mesh(),
        scratch_types=[
            pltpu.VMEM((2, CH), jnp.int32),
            pltpu.VMEM((CH, DEG_W), jnp.float32),
            pltpu.VMEM((ZROWS, DEG_W), jnp.float32),
            pltpu.VMEM_SHARED((N_PAD, DEG_W), jnp.float32),
        ],
    )
    return kern(dst)


SB = 1


def _agg_body(H, esplit, zt_hbm, src0_hbm, src1_hbm, dst_hbm, out_hbm,
              sidx, didx, rows, zbuf, acc, sem_i, sem_g, sem_s):
    c = lax.axis_index("c")
    s = lax.axis_index("s")
    if esplit:
        e_per = E_PAD // (NC * NS)
        base = s * (NC * e_per) + c * (SB * CH)
        stride = NC * SB * CH
    else:
        e_per = E_PAD // NS
        base = s * e_per
        stride = SB * CH
    nsb = e_per // (SB * CH)

    _zero_vmem(zbuf, ZROWS, H)

    def zcopy(k, _):
        pltpu.sync_copy(zbuf, acc.at[pl.ds(s * ROWS_PER_TILE + k * ZROWS, ZROWS)])
        return 0

    lax.fori_loop(0, ROWS_PER_TILE // ZROWS, zcopy, 0)
    plsc.subcore_barrier()

    def load_idx(sb, islot, sem):
        off = base + sb * stride

        @pl.when(c == 0)
        def _():
            pltpu.async_copy(src0_hbm.at[pl.ds(off, SB * CH)], sidx.at[islot, 0], sem)

        @pl.when(c == 1)
        def _():
            pltpu.async_copy(src1_hbm.at[pl.ds(off, SB * CH)], sidx.at[islot, 0], sem)
        for k in range(SB):
            pltpu.async_copy(dst_hbm.at[pl.ds(off + k * CH, CH)],
                             didx.at[islot, k, 0], sem)

    def fire_gathers(islot, rslot):
        for k in range(SB):
            pltpu.async_copy(zt_hbm.at[sidx.at[islot, 0, pl.ds(k * CH, CH)]],
                             rows.at[rslot, k], sem_g)

    def fire_scatters(islot, rslot):
        for k in range(SB):
            pltpu.async_copy(rows.at[rslot, k], acc.at[didx.at[islot, k, 0]],
                             sem_s, add=True)

    def drain_rows(sem, count):
        for _ in range(count):
            pltpu.make_async_copy(zt_hbm.at[pl.ds(0, CH)], rows.at[0, 0], sem).wait()

    def drain_idx(sem, count):
        for _ in range(count):
            pltpu.make_async_copy(src0_hbm.at[pl.ds(0, SB * CH)], sidx.at[0, 0],
                                  sem).wait()
            for k in range(SB):
                pltpu.make_async_copy(dst_hbm.at[pl.ds(0, CH)], didx.at[0, k, 0],
                                      sem).wait()

    load_idx(0, 0, sem_i)
    drain_idx(sem_i, 1)
    fire_gathers(0, 0)

    @pl.when(nsb > 1)
    def _():
        load_idx(1, 1, sem_i)

    def step(g, _):
        rslot = g % 2
        nslot = 1 - rslot
        i_g = g % 3
        i1 = (g + 1) % 3
        i2 = (g + 2) % 3

        drain_rows(sem_g, SB)

        @pl.when(g > 0)
        def _():
            drain_rows(sem_s, SB)

        @pl.when(g + 1 < nsb)
        def _():
            drain_idx(sem_i, 1)
            fire_gathers(i1, nslot)

        fire_scatters(i_g, rslot)

        @pl.when(g + 2 < nsb)
        def _():
            load_idx(g + 2, i2, sem_i)

        return 0

    lax.fori_loop(0, nsb, step, 0)
    drain_rows(sem_s, SB)
    plsc.subcore_barrier()

    r0 = s * ROWS_PER_TILE

    @pl.when(c == 0)
    def _():
        pltpu.sync_copy(acc.at[pl.ds(r0, ROWS_PER_TILE)],
                        out_hbm.at[0, pl.ds(r0, ROWS_PER_TILE)])

    @pl.when(c == 1)
    def _():
        pltpu.sync_copy(acc.at[pl.ds(r0, ROWS_PER_TILE)],
                        out_hbm.at[1, pl.ds(r0, ROWS_PER_TILE)])


def _sc_aggregate(zt_flat, src0, src1, dst, H, esplit):
    kern = pl.kernel(
        functools.partial(_agg_body, H, esplit),
        out_type=jax.ShapeDtypeStruct((NC, N_PAD, H), jnp.float32),
        mesh=_mesh(),
        scratch_types=[
            pltpu.VMEM((3, 1, SB * CH), jnp.int32),
            pltpu.VMEM((3, SB, 1, CH), jnp.int32),
            pltpu.VMEM((2, SB, CH, H), jnp.float32),
            pltpu.VMEM((ZROWS, H), jnp.float32),
            pltpu.VMEM_SHARED((N_PAD, H), jnp.float32),
            pltpu.SemaphoreType.DMA,
            pltpu.SemaphoreType.DMA,
            pltpu.SemaphoreType.DMA,
        ],
    )
    return kern(zt_flat, src0, src1, dst)


def _dinv_from(deg_ref):
    deg = deg_ref[0, :, 0] + deg_ref[1, :, 0] + 1.0
    return lax.rsqrt(deg)[:, None]


def _tc1_body(x_ref, w_ref, o_ref):
    z = jnp.dot(x_ref[...], w_ref[...], preferred_element_type=jnp.float32)
    h = z.shape[1] // 2
    o_ref[0] = z[:, :h]
    o_ref[1] = z[:, h:]


def _tc_layer1(x, W1):
    K = x.shape[1]
    D = W1.shape[1]
    grid = N_NODES // M_BLK
    return pl.pallas_call(
        _tc1_body,
        grid=(grid,),
        in_specs=[
            pl.BlockSpec((M_BLK, K), lambda m: (m, 0)),
            pl.BlockSpec((K, D), lambda m: (0, 0)),
        ],
        out_specs=pl.BlockSpec((2, M_BLK, D // 2), lambda m: (0, m, 0)),
        out_shape=jax.ShapeDtypeStruct((2, N_NODES, D // 2), jnp.float32),
    )(x, W1)


def _tc_scale_body(z_ref, deg_ref, o_ref):
    dinv = _dinv_from(deg_ref)
    o_ref[0] = z_ref[0] * dinv
    o_ref[1] = z_ref[1] * dinv


def _tc_scale(z, deg2):
    H = z.shape[2]
    grid = N_NODES // M_BLK
    return pl.pallas_call(
        _tc_scale_body,
        grid=(grid,),
        in_specs=[
            pl.BlockSpec((2, M_BLK, H), lambda m: (0, m, 0)),
            pl.BlockSpec((2, M_BLK, DEG_W), lambda m: (0, m, 0)),
        ],
        out_specs=pl.BlockSpec((2, M_BLK, H), lambda m: (0, m, 0)),
        out_shape=jax.ShapeDtypeStruct((2, N_NODES, H), jnp.float32),
    )(z, deg2)


def _tc_mid_body(relu, D, D2, split_in, agg_ref, zt_ref, deg_ref, b_ref, w_ref, o_ref):
    dinv = _dinv_from(deg_ref)
    if split_in:
        agg = jnp.concatenate([agg_ref[0], agg_ref[1]], axis=1)
        zt = jnp.concatenate([zt_ref[0], zt_ref[1]], axis=1)
    else:
        agg = (agg_ref[0] + agg_ref[1])[:, :D]
        zt = zt_ref[0][:, :D]
    h = dinv * (agg + zt) + b_ref[...][None, :]
    if relu:
        h = jnp.maximum(h, 0.0)
    z2 = jnp.dot(h, w_ref[...], preferred_element_type=jnp.float32)
    zt2 = z2 * dinv
    if D2 < 128:
        zt2 = jnp.concatenate(
            [zt2, jnp.zeros((zt2.shape[0], 128 - D2), jnp.float32)], axis=1)
    o_ref[0] = zt2
    o_ref[1] = zt2


def _tc_mid(agg, zt, deg2, b, W, relu, split_in):
    D = b.shape[0]
    D2 = W.shape[1]
    grid = N_NODES // M_BLK
    zt_spec = pl.BlockSpec((2, M_BLK, D // 2 if split_in else 128),
                           lambda m: (0, m, 0))
    return pl.pallas_call(
        functools.partial(_tc_mid_body, relu, D, D2, split_in),
        grid=(grid,),
        in_specs=[
            pl.BlockSpec((2, M_BLK, 128), lambda m: (0, m, 0)),
            zt_spec,
            pl.BlockSpec((2, M_BLK, DEG_W), lambda m: (0, m, 0)),
            pl.BlockSpec((D,), lambda m: (0,)),
            pl.BlockSpec((D, D2), lambda m: (0, 0)),
        ],
        out_specs=pl.BlockSpec((2, M_BLK, 128), lambda m: (0, m, 0)),
        out_shape=jax.ShapeDtypeStruct((2, N_NODES, 128), jnp.float32),
    )(agg, zt, deg2, b, W)


def _tc_final_body(agg_ref, zt_ref, deg_ref, b_ref, wl_ref, bl_ref, h_ref, o_ref):
    D = b_ref.shape[0]
    dinv = _dinv_from(deg_ref)
    agg = (agg_ref[0] + agg_ref[1])[:, :D]
    zt = zt_ref[0][:, :D]
    h3 = dinv * (agg + zt) + b_ref[...][None, :]
    h_ref[...] = h3
    logits = jnp.dot(h3, wl_ref[...], preferred_element_type=jnp.float32)
    o_ref[...] = jax.nn.sigmoid(logits + bl_ref[0])


def _tc_final(agg, zt, deg2, b3, Wl, bl):
    D = b3.shape[0]
    grid = N_NODES // M_BLK
    return pl.pallas_call(
        _tc_final_body,
        grid=(grid,),
        in_specs=[
            pl.BlockSpec((2, M_BLK, 128), lambda m: (0, m, 0)),
            pl.BlockSpec((2, M_BLK, 128), lambda m: (0, m, 0)),
            pl.BlockSpec((2, M_BLK, DEG_W), lambda m: (0, m, 0)),
            pl.BlockSpec((D,), lambda m: (0,)),
            pl.BlockSpec((D, 1), lambda m: (0, 0)),
            pl.BlockSpec((1,), lambda m: (0,)),
        ],
        out_specs=[
            pl.BlockSpec((M_BLK, D), lambda m: (m, 0)),
            pl.BlockSpec((M_BLK, 1), lambda m: (m, 0)),
        ],
        out_shape=[
            jax.ShapeDtypeStruct((N_NODES, D), jnp.float32),
            jax.ShapeDtypeStruct((N_NODES, 1), jnp.float32),
        ],
    )(agg, zt, deg2, b3, Wl, bl)


def kernel(x, W1, b1, W2, b2, W3, b3, Wl, bl, edge_index):
    pad = E_PAD - N_EDGES
    src = jnp.concatenate([edge_index[0], jnp.zeros((pad,), jnp.int32)])
    dst = jnp.concatenate([edge_index[1], jnp.full((pad,), N_NODES, jnp.int32)])
    srcN = src + N_NODES

    z1 = _tc_layer1(x, W1)
    deg2 = _sc_degree(dst)
    zt1 = _tc_scale(z1, deg2)
    agg1 = _sc_aggregate(zt1.reshape(2 * N_NODES, 128), src, srcN, dst,
                         128, esplit=False)

    zt2 = _tc_mid(agg1, zt1, deg2, b1, W2, relu=True,
                  split_in=True)
    agg2 = _sc_aggregate(zt2.reshape(2 * N_NODES, 128), src, srcN, dst,
                         128, esplit=True)

    zt3 = _tc_mid(agg2, zt2, deg2, b2, W3, relu=False,
                  split_in=False)
    agg3 = _sc_aggregate(zt3.reshape(2 * N_NODES, 128), src, srcN, dst,
                         128, esplit=True)

    h3, out = _tc_final(agg3, zt3, deg2, b3, Wl, bl)
    return (out, h3)
